# baseline (speedup 1.0000x reference)
import functools

import jax
import jax.numpy as jnp
from jax import lax
from jax.experimental import pallas as pl
from jax.experimental.pallas import tpu as pltpu
from jax.experimental.pallas import tpu_sc as plsc

B, L, V, E, H = 128, 20, 100000, 256, 512
BL = B * L

_NC, _NS = 2, 16
_NW = _NC * _NS
_B_PER_W = BL // _NW

@functools.lru_cache(maxsize=1)
def _sc_gather_fn():
    mesh = plsc.VectorSubcoreMesh(core_axis_name="c", subcore_axis_name="s")

    @functools.partial(
        pl.kernel,
        mesh=mesh,
        out_type=jax.ShapeDtypeStruct((BL, E), jnp.float32),
        scratch_types=[
            pltpu.VMEM((_B_PER_W,), jnp.int32),
            pltpu.VMEM((_B_PER_W, E), jnp.float32),
            pltpu.SemaphoreType.DMA,
        ],
    )
    def _sc_gather(table_hbm, idx_hbm, out_hbm, idx_v, rows_v, sem):
        wid = lax.axis_index("s") * _NC + lax.axis_index("c")
        base = wid * _B_PER_W
        pltpu.sync_copy(idx_hbm.at[pl.ds(base, _B_PER_W)], idx_v)
        pltpu.async_copy(table_hbm.at[idx_v], rows_v, sem).wait()
        pltpu.sync_copy(rows_v, out_hbm.at[pl.ds(base, _B_PER_W)])

    return _sc_gather


def _lstm_body(emb_ref, h0_ref, c0_ref, wihT_ref, whhT_ref, bias_ref,
               hs_ref, hT_ref, cT_ref, h_sc, c_sc):
    h_sc[...] = h0_ref[...]
    c_sc[...] = c0_ref[...]

    def step(t, carry):
        x_t = emb_ref[pl.ds(t * B, B), :]
        gates = (
            jnp.dot(x_t, wihT_ref[...], preferred_element_type=jnp.float32)
            + jnp.dot(h_sc[...], whhT_ref[...], preferred_element_type=jnp.float32)
            + bias_ref[...]
        )
        i = jax.nn.sigmoid(gates[:, 0:H])
        f = jax.nn.sigmoid(gates[:, H:2 * H])
        g = jnp.tanh(gates[:, 2 * H:3 * H])
        o = jax.nn.sigmoid(gates[:, 3 * H:4 * H])
        c_new = f * c_sc[...] + i * g
        h_new = o * jnp.tanh(c_new)
        c_sc[...] = c_new
        h_sc[...] = h_new
        hs_ref[pl.ds(t * B, B), :] = h_new.astype(jnp.bfloat16)
        return carry

    lax.fori_loop(0, L, step, 0)
    hT_ref[...] = h_sc[...]
    cT_ref[...] = c_sc[...]


def _lstm_call(emb, h0, c0, wihT, whhT, bias):
    return pl.pallas_call(
        _lstm_body,
        out_shape=(
            jax.ShapeDtypeStruct((BL, H), jnp.bfloat16),
            jax.ShapeDtypeStruct((B, H), jnp.float32),
            jax.ShapeDtypeStruct((B, H), jnp.float32),
        ),
        scratch_shapes=[
            pltpu.VMEM((B, H), jnp.float32),
            pltpu.VMEM((B, H), jnp.float32),
        ],
    )(emb, h0, c0, wihT, whhT, bias)


_VB = 512
_NVB = (V + _VB - 1) // _VB


def _fc_body(x_ref, w_ref, b_ref, o_ref):
    w = w_ref[...].astype(jnp.bfloat16)
    acc = lax.dot_general(
        x_ref[...], w,
        dimension_numbers=(((1,), (1,)), ((), ())),
        preferred_element_type=jnp.float32,
    )
    o_ref[...] = acc + b_ref[...]


def _fc_call(x_bf, fc_w, fc_b2d):
    return pl.pallas_call(
        _fc_body,
        grid=(_NVB,),
        in_specs=[
            pl.BlockSpec((BL, H), lambda v: (0, 0)),
            pl.BlockSpec((_VB, H), lambda v: (v, 0)),
            pl.BlockSpec((1, _VB), lambda v: (0, v)),
        ],
        out_specs=pl.BlockSpec((BL, _VB), lambda v: (0, v)),
        out_shape=jax.ShapeDtypeStruct((BL, V), jnp.float32),
    )(x_bf, fc_w, fc_b2d)


def kernel(y_before, h0, c0, embed_w, w_ih, w_hh, b_ih, b_hh, fc_w, fc_b):
    idx = y_before.astype(jnp.int32).T.reshape(-1)
    emb = _sc_gather_fn()(embed_w, idx)

    wihT = w_ih.T
    whhT = w_hh.T
    bias = (b_ih + b_hh).reshape(1, 4 * H)
    hs, hT, cT = _lstm_call(emb, h0[0], c0[0], wihT, whhT, bias)

    x_bf = jnp.swapaxes(hs.reshape(L, B, H), 0, 1).reshape(BL, H)
    out = _fc_call(x_bf, fc_w, fc_b.reshape(1, V))
    return out.reshape(B, L, V), hT[None, ...], cT[None, ...]


# speedup vs baseline: 2.6454x; 2.6454x over previous
import functools

import jax
import jax.numpy as jnp
from jax import lax
from jax.experimental import pallas as pl
from jax.experimental.pallas import tpu as pltpu
from jax.experimental.pallas import tpu_sc as plsc

B, L, V, E, H = 128, 20, 100000, 256, 512
BL = B * L

_NC, _NS = 2, 16
_NW = _NC * _NS
_B_PER_W = BL // _NW


@functools.lru_cache(maxsize=1)
def _sc_gather_fn():
    mesh = plsc.VectorSubcoreMesh(core_axis_name="c", subcore_axis_name="s")

    @functools.partial(
        pl.kernel,
        mesh=mesh,
        out_type=jax.ShapeDtypeStruct((BL, E), jnp.float32),
        scratch_types=[
            pltpu.VMEM((_B_PER_W,), jnp.int32),
            pltpu.VMEM((_B_PER_W, E), jnp.float32),
            pltpu.SemaphoreType.DMA,
        ],
    )
    def _sc_gather(table_hbm, idx_hbm, out_hbm, idx_v, rows_v, sem):
        wid = lax.axis_index("s") * _NC + lax.axis_index("c")
        base = wid * _B_PER_W
        pltpu.sync_copy(idx_hbm.at[pl.ds(base, _B_PER_W)], idx_v)
        pltpu.async_copy(table_hbm.at[idx_v], rows_v, sem).wait()
        pltpu.sync_copy(rows_v, out_hbm.at[pl.ds(base, _B_PER_W)])

    return _sc_gather


def _lstm_body(emb_ref, h0_ref, c0_ref, wih_ref, whh_ref, biasT_ref,
               hsT_ref, hT_ref, cT_ref, h_sc, c_sc):
    h_sc[...] = h0_ref[...].T
    c_sc[...] = c0_ref[...].T

    def step(t, carry):
        xT = emb_ref[pl.ds(t * B, B), :].T
        gates = (
            jnp.dot(wih_ref[...], xT, preferred_element_type=jnp.float32)
            + jnp.dot(whh_ref[...], h_sc[...], preferred_element_type=jnp.float32)
            + biasT_ref[...]
        )
        i = jax.nn.sigmoid(gates[0:H, :])
        f = jax.nn.sigmoid(gates[H:2 * H, :])
        g = jnp.tanh(gates[2 * H:3 * H, :])
        o = jax.nn.sigmoid(gates[3 * H:4 * H, :])
        c_new = f * c_sc[...] + i * g
        h_new = o * jnp.tanh(c_new)
        c_sc[...] = c_new
        h_sc[...] = h_new
        hsT_ref[pl.ds(t, 1), :, :] = h_new.astype(jnp.bfloat16)[None, ...]
        return carry

    lax.fori_loop(0, L, step, 0)
    hT_ref[...] = h_sc[...].T
    cT_ref[...] = c_sc[...].T


def _lstm_call(emb, h0, c0, wih, whh, biasT):
    return pl.pallas_call(
        _lstm_body,
        out_shape=(
            jax.ShapeDtypeStruct((L, H, B), jnp.bfloat16),
            jax.ShapeDtypeStruct((B, H), jnp.float32),
            jax.ShapeDtypeStruct((B, H), jnp.float32),
        ),
        scratch_shapes=[
            pltpu.VMEM((H, B), jnp.float32),
            pltpu.VMEM((H, B), jnp.float32),
        ],
    )(emb, h0, c0, wih, whh, biasT)


_VB = 1024
_NVB = (V + _VB - 1) // _VB
_LG = 4
_NLG = L // _LG


def _fc_body(xT_ref, w_ref, b_ref, o_ref):
    g = pl.program_id(1)
    xtg = jnp.concatenate(
        [xT_ref[_LG * g + k] for k in range(_LG)], axis=1)
    acc = lax.dot_general(
        w_ref[...], xtg,
        dimension_numbers=(((1,), (0,)), ((), ())),
        precision=lax.Precision.DEFAULT,
        preferred_element_type=jnp.float32,
    )
    bias = b_ref[...]
    for k in range(_LG):
        o_ref[k] = acc[:, k * B:(k + 1) * B] + bias


def _fc_call(hsT, fc_w, fc_b2d):
    return pl.pallas_call(
        _fc_body,
        grid=(_NVB, _NLG),
        in_specs=[
            pl.BlockSpec((L, H, B), lambda v, g: (0, 0, 0)),
            pl.BlockSpec((_VB, H), lambda v, g: (v, 0)),
            pl.BlockSpec((_VB, 1), lambda v, g: (v, 0)),
        ],
        out_specs=pl.BlockSpec((_LG, _VB, B), lambda v, g: (g, v, 0)),
        out_shape=jax.ShapeDtypeStruct((L, V, B), jnp.float32),
    )(hsT, fc_w, fc_b2d)


def kernel(y_before, h0, c0, embed_w, w_ih, w_hh, b_ih, b_hh, fc_w, fc_b):
    idx = y_before.astype(jnp.int32).T.reshape(-1)
    emb = _sc_gather_fn()(embed_w, idx)

    biasT = (b_ih + b_hh).reshape(4 * H, 1)
    hsT, hT, cT = _lstm_call(emb, h0[0], c0[0], w_ih, w_hh, biasT)

    out_t = _fc_call(hsT, fc_w, fc_b.reshape(V, 1))
    dec_fc = jnp.transpose(out_t, (2, 0, 1))
    return dec_fc, hT[None, ...], cT[None, ...]


# speedup vs baseline: 3.1360x; 1.1855x over previous
import functools

import jax
import jax.numpy as jnp
from jax import lax
from jax.experimental import pallas as pl
from jax.experimental.pallas import tpu as pltpu
from jax.experimental.pallas import tpu_sc as plsc

B, L, V, E, H = 128, 20, 100000, 256, 512
BL = B * L

_NC, _NS = 2, 16
_NW = _NC * _NS
_B_PER_W = BL // _NW


@functools.lru_cache(maxsize=1)
def _sc_gather_fn():
    mesh = plsc.VectorSubcoreMesh(core_axis_name="c", subcore_axis_name="s")

    @functools.partial(
        pl.kernel,
        mesh=mesh,
        out_type=jax.ShapeDtypeStruct((BL, E), jnp.float32),
        scratch_types=[
            pltpu.VMEM((_B_PER_W,), jnp.int32),
            pltpu.VMEM((_B_PER_W, E), jnp.float32),
            pltpu.SemaphoreType.DMA,
        ],
    )
    def _sc_gather(table_hbm, idx_hbm, out_hbm, idx_v, rows_v, sem):
        wid = lax.axis_index("s") * _NC + lax.axis_index("c")
        base = wid * _B_PER_W
        pltpu.sync_copy(idx_hbm.at[pl.ds(base, _B_PER_W)], idx_v)
        pltpu.async_copy(table_hbm.at[idx_v], rows_v, sem).wait()
        pltpu.sync_copy(rows_v, out_hbm.at[pl.ds(base, _B_PER_W)])

    return _sc_gather


def _lstm_body(emb_ref, h0_ref, c0_ref, wih_ref, whh_ref, biasT_ref,
               hsT_ref, hT_ref, cT_ref, h_sc, c_sc):
    h_sc[...] = h0_ref[...].T
    c_sc[...] = c0_ref[...].T

    def step(t, carry):
        xT = emb_ref[pl.ds(t * B, B), :].T
        gates = (
            jnp.dot(wih_ref[...], xT, preferred_element_type=jnp.float32)
            + jnp.dot(whh_ref[...], h_sc[...], preferred_element_type=jnp.float32)
            + biasT_ref[...]
        )
        i = jax.nn.sigmoid(gates[0:H, :])
        f = jax.nn.sigmoid(gates[H:2 * H, :])
        g = jnp.tanh(gates[2 * H:3 * H, :])
        o = jax.nn.sigmoid(gates[3 * H:4 * H, :])
        c_new = f * c_sc[...] + i * g
        h_new = o * jnp.tanh(c_new)
        c_sc[...] = c_new
        h_sc[...] = h_new
        hsT_ref[pl.ds(t, 1), :, :] = h_new.astype(jnp.bfloat16)[None, ...]
        return carry

    lax.fori_loop(0, L, step, 0)
    hT_ref[...] = h_sc[...].T
    cT_ref[...] = c_sc[...].T


def _lstm_call(emb, h0, c0, wih, whh, biasT):
    return pl.pallas_call(
        _lstm_body,
        out_shape=(
            jax.ShapeDtypeStruct((L, H, B), jnp.bfloat16),
            jax.ShapeDtypeStruct((B, H), jnp.float32),
            jax.ShapeDtypeStruct((B, H), jnp.float32),
        ),
        scratch_shapes=[
            pltpu.VMEM((H, B), jnp.float32),
            pltpu.VMEM((H, B), jnp.float32),
        ],
    )(emb, h0, c0, wih, whh, biasT)


_VB = 2048
_NVB = (V + _VB - 1) // _VB
_LG = 4
_NLG = L // _LG


def _fc_body(xT_ref, w_ref, b_ref, o_ref):
    g = pl.program_id(1)
    xtg = jnp.concatenate(
        [xT_ref[_LG * g + k] for k in range(_LG)], axis=1)
    acc = lax.dot_general(
        w_ref[...], xtg,
        dimension_numbers=(((1,), (0,)), ((), ())),
        precision=lax.Precision.DEFAULT,
        preferred_element_type=jnp.float32,
    )
    bias = b_ref[...]
    for k in range(_LG):
        o_ref[k] = acc[:, k * B:(k + 1) * B] + bias


def _fc_call(hsT, fc_w, fc_b2d):
    return pl.pallas_call(
        _fc_body,
        grid=(_NVB, _NLG),
        in_specs=[
            pl.BlockSpec((L, H, B), lambda v, g: (0, 0, 0)),
            pl.BlockSpec((_VB, H), lambda v, g: (v, 0)),
            pl.BlockSpec((_VB, 1), lambda v, g: (v, 0)),
        ],
        out_specs=pl.BlockSpec((_LG, _VB, B), lambda v, g: (g, v, 0)),
        out_shape=jax.ShapeDtypeStruct((L, V, B), jnp.float32),
    )(hsT, fc_w, fc_b2d)


def kernel(y_before, h0, c0, embed_w, w_ih, w_hh, b_ih, b_hh, fc_w, fc_b):
    idx = y_before.astype(jnp.int32).T.reshape(-1)
    emb = _sc_gather_fn()(embed_w, idx)

    biasT = (b_ih + b_hh).reshape(4 * H, 1)
    hsT, hT, cT = _lstm_call(emb, h0[0], c0[0], w_ih, w_hh, biasT)

    out_t = _fc_call(hsT, fc_w, fc_b.reshape(V, 1))
    dec_fc = jnp.transpose(out_t, (2, 0, 1))
    return dec_fc, hT[None, ...], cT[None, ...]


# speedup vs baseline: 3.4951x; 1.1145x over previous
import functools

import jax
import jax.numpy as jnp
from jax import lax
from jax.experimental import pallas as pl
from jax.experimental.pallas import tpu as pltpu
from jax.experimental.pallas import tpu_sc as plsc

B, L, V, E, H = 128, 20, 100000, 256, 512
BL = B * L

_NC, _NS = 2, 16
_NW = _NC * _NS
_B_PER_W = BL // _NW


@functools.lru_cache(maxsize=1)
def _sc_gather_fn():
    mesh = plsc.VectorSubcoreMesh(core_axis_name="c", subcore_axis_name="s")

    @functools.partial(
        pl.kernel,
        mesh=mesh,
        out_type=jax.ShapeDtypeStruct((BL, E), jnp.float32),
        scratch_types=[
            pltpu.VMEM((_B_PER_W,), jnp.int32),
            pltpu.VMEM((_B_PER_W, E), jnp.float32),
            pltpu.SemaphoreType.DMA,
        ],
    )
    def _sc_gather(table_hbm, idx_hbm, out_hbm, idx_v, rows_v, sem):
        wid = lax.axis_index("s") * _NC + lax.axis_index("c")
        base = wid * _B_PER_W
        pltpu.sync_copy(idx_hbm.at[pl.ds(base, _B_PER_W)], idx_v)
        pltpu.async_copy(table_hbm.at[idx_v], rows_v, sem).wait()
        pltpu.sync_copy(rows_v, out_hbm.at[pl.ds(base, _B_PER_W)])

    return _sc_gather


def _lstm_body(emb_ref, h0_ref, c0_ref, wih_ref, whh_ref, biasT_ref,
               hsT_ref, hT_ref, cT_ref, h_sc, c_sc):
    h_sc[...] = h0_ref[...].T
    c_sc[...] = c0_ref[...].T

    def step(t, carry):
        xT = emb_ref[pl.ds(t * B, B), :].T
        gates = (
            jnp.dot(wih_ref[...], xT, preferred_element_type=jnp.float32)
            + jnp.dot(whh_ref[...], h_sc[...], preferred_element_type=jnp.float32)
            + biasT_ref[...]
        )
        i = jax.nn.sigmoid(gates[0:H, :])
        f = jax.nn.sigmoid(gates[H:2 * H, :])
        g = jnp.tanh(gates[2 * H:3 * H, :])
        o = jax.nn.sigmoid(gates[3 * H:4 * H, :])
        c_new = f * c_sc[...] + i * g
        h_new = o * jnp.tanh(c_new)
        c_sc[...] = c_new
        h_sc[...] = h_new
        hsT_ref[pl.ds(t, 1), :, :] = h_new.astype(jnp.bfloat16)[None, ...]
        return carry

    lax.fori_loop(0, L, step, 0)
    hT_ref[...] = h_sc[...].T
    cT_ref[...] = c_sc[...].T


def _lstm_call(emb, h0, c0, wih, whh, biasT):
    return pl.pallas_call(
        _lstm_body,
        out_shape=(
            jax.ShapeDtypeStruct((L, H, B), jnp.bfloat16),
            jax.ShapeDtypeStruct((B, H), jnp.float32),
            jax.ShapeDtypeStruct((B, H), jnp.float32),
        ),
        scratch_shapes=[
            pltpu.VMEM((H, B), jnp.float32),
            pltpu.VMEM((H, B), jnp.float32),
        ],
    )(emb, h0, c0, wih, whh, biasT)


_VB = 4096
_NVB = (V + _VB - 1) // _VB
_LG = 4
_NLG = L // _LG


def _fc_body(xT_ref, w_ref, b_ref, o_ref):
    g = pl.program_id(1)
    xtg = jnp.concatenate(
        [xT_ref[_LG * g + k] for k in range(_LG)], axis=1)
    acc = lax.dot_general(
        w_ref[...], xtg,
        dimension_numbers=(((1,), (0,)), ((), ())),
        precision=lax.Precision.DEFAULT,
        preferred_element_type=jnp.float32,
    )
    bias = b_ref[...]
    for k in range(_LG):
        o_ref[k] = acc[:, k * B:(k + 1) * B] + bias


def _fc_call(hsT, fc_w, fc_b2d):
    return pl.pallas_call(
        _fc_body,
        grid=(_NVB, _NLG),
        in_specs=[
            pl.BlockSpec((L, H, B), lambda v, g: (0, 0, 0)),
            pl.BlockSpec((_VB, H), lambda v, g: (v, 0)),
            pl.BlockSpec((_VB, 1), lambda v, g: (v, 0)),
        ],
        out_specs=pl.BlockSpec((_LG, _VB, B), lambda v, g: (g, v, 0)),
        out_shape=jax.ShapeDtypeStruct((L, V, B), jnp.float32),
    )(hsT, fc_w, fc_b2d)


def kernel(y_before, h0, c0, embed_w, w_ih, w_hh, b_ih, b_hh, fc_w, fc_b):
    idx = y_before.astype(jnp.int32).T.reshape(-1)
    emb = _sc_gather_fn()(embed_w, idx)

    biasT = (b_ih + b_hh).reshape(4 * H, 1)
    hsT, hT, cT = _lstm_call(emb, h0[0], c0[0], w_ih, w_hh, biasT)

    out_t = _fc_call(hsT, fc_w, fc_b.reshape(V, 1))
    dec_fc = jnp.transpose(out_t, (2, 0, 1))
    return dec_fc, hT[None, ...], cT[None, ...]
